# baseline (speedup 1.0000x reference)
import jax
import jax.numpy as jnp
from jax.experimental import pallas as pl
from jax.experimental.pallas import tpu as pltpu

_R = 32
_NBUF = 8


def _make_body(n_steps, rows):
    def body(x_hbm, pe_ref, o_hbm, xb, ob, in_sems, out_sems):
        i = pl.program_id(0)
        slot = jax.lax.rem(i, _NBUF)

        def in_copy(step, slot_):
            return pltpu.make_async_copy(
                x_hbm.at[pl.ds(step * rows, rows), :, :],
                xb.at[slot_],
                in_sems.at[slot_],
            )

        def out_copy(step, slot_):
            return pltpu.make_async_copy(
                ob.at[slot_],
                o_hbm.at[pl.ds(step * rows, rows), :, :],
                out_sems.at[slot_],
            )

        @pl.when(i == 0)
        def _prologue():
            for j in range(_NBUF):
                in_copy(j, j).start()

        @pl.when(i >= _NBUF)
        def _wait_prev_out():
            out_copy(i - _NBUF, slot).wait()

        in_copy(i, slot).wait()
        ob[slot] = xb[slot] + pe_ref[...]
        out_copy(i, slot).start()

        @pl.when(i + _NBUF < n_steps)
        def _next_in():
            in_copy(i + _NBUF, slot).start()

        @pl.when(i == n_steps - 1)
        def _epilogue():
            for j in range(_NBUF):
                step = n_steps - _NBUF + j
                out_copy(step, step % _NBUF).wait()

    return body


def kernel(x, pos_emb):
    B, T, D = x.shape
    n_steps = B // _R
    return pl.pallas_call(
        _make_body(n_steps, _R),
        grid=(n_steps,),
        in_specs=[
            pl.BlockSpec(memory_space=pl.ANY),
            pl.BlockSpec((1, T, D), lambda i: (0, 0, 0)),
        ],
        out_specs=pl.BlockSpec(memory_space=pl.ANY),
        out_shape=jax.ShapeDtypeStruct((B, T, D), x.dtype),
        scratch_shapes=[
            pltpu.MemorySpace.VMEM((_NBUF, _R, T, D), jnp.float32),
            pltpu.MemorySpace.VMEM((_NBUF, _R, T, D), jnp.float32),
            pltpu.SemaphoreType.DMA((_NBUF,)),
            pltpu.SemaphoreType.DMA((_NBUF,)),
        ],
    )(x, pos_emb.reshape(1, T, D))


# speedup vs baseline: 5.8751x; 5.8751x over previous
import jax
import jax.numpy as jnp
from jax.experimental import pallas as pl

_RR = 512


def _add_kernel(x_ref, pe_ref, o_ref):
    o_ref[...] = x_ref[...] + pe_ref[...]


def kernel(x, pos_emb):
    B, T, D = x.shape
    N = T * D
    xt = x.transpose(1, 2, 0).reshape(N, B)
    pe = pos_emb.reshape(N, 1)
    out = pl.pallas_call(
        _add_kernel,
        grid=(N // _RR,),
        in_specs=[
            pl.BlockSpec((_RR, B), lambda i: (i, 0)),
            pl.BlockSpec((_RR, 1), lambda i: (i, 0)),
        ],
        out_specs=pl.BlockSpec((_RR, B), lambda i: (i, 0)),
        out_shape=jax.ShapeDtypeStruct((N, B), x.dtype),
    )(xt, pe)
    return out.reshape(T, D, B).transpose(2, 0, 1)
